# baseline (speedup 1.0000x reference)
import functools

import jax
import jax.numpy as jnp
from jax import lax
from jax.experimental import pallas as pl
from jax.experimental.pallas import tpu as pltpu
from jax.experimental.pallas import tpu_sc as plsc

_ROWS, _WIDTH = 8192, 2048
_NC, _NS = 2, 16
_NW = _NC * _NS
_ROWS_PER_W = _ROWS // _NW

_mesh = plsc.VectorSubcoreMesh(core_axis_name="c", subcore_axis_name="s")


@functools.partial(
    pl.kernel,
    mesh=_mesh,
    out_type=jax.ShapeDtypeStruct((_ROWS, _WIDTH), jnp.float32),
    scratch_types=[pltpu.SemaphoreType.DMA],
)
def _sc_copy(src_hbm, dst_hbm, sem):
    wid = lax.axis_index("s") * _NC + lax.axis_index("c")
    base = wid * _ROWS_PER_W
    pltpu.make_async_copy(
        src_hbm.at[pl.ds(base, _ROWS_PER_W)],
        dst_hbm.at[pl.ds(base, _ROWS_PER_W)],
        sem,
    ).start()
    pltpu.make_async_copy(
        src_hbm.at[pl.ds(base, _ROWS_PER_W)],
        dst_hbm.at[pl.ds(base, _ROWS_PER_W)],
        sem,
    ).wait()


def kernel(pos_emb):
    return _sc_copy(pos_emb)


# speedup vs baseline: 30.3907x; 30.3907x over previous
import functools

import jax
import jax.numpy as jnp
from jax import lax
from jax.experimental import pallas as pl
from jax.experimental.pallas import tpu as pltpu
from jax.experimental.pallas import tpu_sc as plsc

_ROWS, _WIDTH = 8192, 2048
_NC, _NS = 2, 16
_NW = _NC * _NS
_ROWS_PER_W = _ROWS // _NW
_CH = 16
_NCHUNK = _ROWS_PER_W // _CH

_mesh = plsc.VectorSubcoreMesh(core_axis_name="c", subcore_axis_name="s")


@functools.partial(
    pl.kernel,
    mesh=_mesh,
    out_type=jax.ShapeDtypeStruct((_ROWS, _WIDTH), jnp.float32),
    scratch_types=[
        pltpu.VMEM((_CH, _WIDTH), jnp.float32),
        pltpu.VMEM((_CH, _WIDTH), jnp.float32),
        pltpu.SemaphoreType.DMA,
        pltpu.SemaphoreType.DMA,
    ],
)
def _sc_copy(src_hbm, dst_hbm, buf0, buf1, rsem, wsem):
    wid = lax.axis_index("s") * _NC + lax.axis_index("c")
    base = wid * _ROWS_PER_W
    bufs = (buf0, buf1)

    def _read(g, buf):
        return pltpu.make_async_copy(
            src_hbm.at[pl.ds(base + g * _CH, _CH)], buf, rsem)

    def _write(g, buf):
        return pltpu.make_async_copy(
            buf, dst_hbm.at[pl.ds(base + g * _CH, _CH)], wsem)

    for g in range(_NCHUNK):
        buf = bufs[g % 2]
        if g >= 2:
            _write(g - 2, buf).wait()
        _read(g, buf).start()
        _read(g, buf).wait()
        _write(g, buf).start()
    _write(_NCHUNK - 2, bufs[_NCHUNK % 2]).wait()
    _write(_NCHUNK - 1, bufs[(_NCHUNK - 1) % 2]).wait()


def kernel(pos_emb):
    return _sc_copy(pos_emb)


# speedup vs baseline: 49.0323x; 1.6134x over previous
import jax
import jax.numpy as jnp
from jax.experimental import pallas as pl
from jax.experimental.pallas import tpu as pltpu

_ROWS, _WIDTH = 8192, 2048
_CH = 256
_N = _ROWS // _CH
_K = 8
_D = 4


def _copy_kernel(src_hbm, dst_hbm, *args):
    bufs = args[:_K]
    rsem, wsem = args[_K], args[_K + 1]

    def _read(g):
        b = g % _K
        return pltpu.make_async_copy(
            src_hbm.at[pl.ds(g * _CH, _CH)], bufs[b], rsem.at[b])

    def _write(g):
        b = g % _K
        return pltpu.make_async_copy(
            bufs[b], dst_hbm.at[pl.ds(g * _CH, _CH)], wsem.at[b])

    for g in range(_N + _D):
        if g < _N:
            if g >= _K:
                _write(g - _K).wait()
            _read(g).start()
        if g >= _D:
            _read(g - _D).wait()
            _write(g - _D).start()
    for g in range(_N - _K, _N):
        _write(g).wait()


def kernel(pos_emb):
    return pl.pallas_call(
        _copy_kernel,
        out_shape=jax.ShapeDtypeStruct((_ROWS, _WIDTH), jnp.float32),
        in_specs=[pl.BlockSpec(memory_space=pl.ANY)],
        out_specs=pl.BlockSpec(memory_space=pl.ANY),
        scratch_shapes=(
            [pltpu.VMEM((_CH, _WIDTH), jnp.float32) for _ in range(_K)]
            + [pltpu.SemaphoreType.DMA((_K,)), pltpu.SemaphoreType.DMA((_K,))]
        ),
    )(pos_emb)


# speedup vs baseline: 49.2979x; 1.0054x over previous
import jax
import jax.numpy as jnp
from jax.experimental import pallas as pl
from jax.experimental.pallas import tpu as pltpu

_BLOCK_ROWS = 1024


def _copy_kernel(src_ref, dst_ref):
    dst_ref[...] = src_ref[...]


def kernel(pos_emb):
    rows, width = pos_emb.shape
    grid = (rows // _BLOCK_ROWS,)
    return pl.pallas_call(
        _copy_kernel,
        out_shape=jax.ShapeDtypeStruct(pos_emb.shape, pos_emb.dtype),
        grid=grid,
        in_specs=[pl.BlockSpec((_BLOCK_ROWS, width), lambda i: (i, 0))],
        out_specs=pl.BlockSpec((_BLOCK_ROWS, width), lambda i: (i, 0)),
    )(pos_emb)


# speedup vs baseline: 49.5356x; 1.0048x over previous
import jax
import jax.numpy as jnp
from jax.experimental import pallas as pl
from jax.experimental.pallas import tpu as pltpu

_ROWS, _WIDTH = 8192, 2048
_CH = 1024
_N = _ROWS // _CH
_K = 3
_D = 1


def _copy_kernel(src_hbm, dst_hbm, *args):
    bufs = args[:_K]
    rsem, wsem = args[_K], args[_K + 1]

    def _read(g):
        b = g % _K
        return pltpu.make_async_copy(
            src_hbm.at[pl.ds(g * _CH, _CH)], bufs[b], rsem.at[b])

    def _write(g):
        b = g % _K
        return pltpu.make_async_copy(
            bufs[b], dst_hbm.at[pl.ds(g * _CH, _CH)], wsem.at[b])

    for g in range(_N + _D):
        if g < _N:
            if g >= _K:
                _write(g - _K).wait()
            _read(g).start()
        if g >= _D:
            _read(g - _D).wait()
            _write(g - _D).start()
    for g in range(_N - _K, _N):
        _write(g).wait()


def kernel(pos_emb):
    return pl.pallas_call(
        _copy_kernel,
        out_shape=jax.ShapeDtypeStruct((_ROWS, _WIDTH), jnp.float32),
        in_specs=[pl.BlockSpec(memory_space=pl.ANY)],
        out_specs=pl.BlockSpec(memory_space=pl.ANY),
        scratch_shapes=(
            [pltpu.VMEM((_CH, _WIDTH), jnp.float32) for _ in range(_K)]
            + [pltpu.SemaphoreType.DMA((_K,)), pltpu.SemaphoreType.DMA((_K,))]
        ),
    )(pos_emb)
